# Initial kernel scaffold; baseline (speedup 1.0000x reference)
import jax
import jax.numpy as jnp
from jax.experimental import pallas as pl


def kernel(upDown_count_T, W, b):
    raise NotImplementedError("write your pallas kernel here")


# speedup vs baseline: 1.0232x; 1.0232x over previous
import functools
import jax
import jax.numpy as jnp
from jax import lax
from jax.experimental import pallas as pl
from jax.experimental.pallas import tpu as pltpu
from jax.experimental.pallas import tpu_sc as plsc

_NROWS = 970
_LANES = 16
_NCORES = 2
_NSUB = 16
_NWORK = _NCORES * _NSUB
_CHUNKS = (_NROWS + _LANES - 1) // _LANES
_NPAD = _CHUNKS * _LANES
_CBUF = _NPAD + 8
_NEG = float(jnp.finfo(jnp.float32).min)

_TOT = _NROWS * _NROWS
_CONVW = _TOT + 28
_CB = 16384
_GRID = -(-_CONVW // _CB)


def _conv_body(b_sm, w_ref, x_ref, o_ref):
    res = lax.dot_general(w_ref[...], x_ref[...], (((1,), (0,)), ((), ())),
                          preferred_element_type=jnp.float32)
    o_ref[...] = res + b_sm[0]


def _conv_tc(x2, wrow, b):
    return pl.pallas_call(
        _conv_body,
        grid=(_GRID,),
        in_specs=[
            pl.BlockSpec(memory_space=pltpu.SMEM),
            pl.BlockSpec((1, 4), lambda i: (0, 0)),
            pl.BlockSpec((4, _CB), lambda i: (0, i)),
        ],
        out_specs=pl.BlockSpec((1, _CB), lambda i: (0, i)),
        out_shape=jax.ShapeDtypeStruct((1, _CONVW), jnp.float32),
    )(b, wrow, x2)


def _topk_body(conv_hbm, out_hbm, rowbuf, stage):
    cid = lax.axis_index("c")
    sid = lax.axis_index("s")
    wid = sid * _NCORES + cid
    lane = lax.iota(jnp.int32, _LANES)

    nrows = jnp.where(wid < _NROWS - 30 * _NWORK, 31, 30)

    def row_body(t, _):
        row = wid + _NWORK * t
        off = row * _NROWS
        start = (off // 8) * 8
        shift = off - start
        pltpu.sync_copy(conv_hbm.at[pl.ds(start, _NPAD)],
                        rowbuf.at[pl.ds(0, _NPAD)])

        def chunk_body(j, carry):
            tv, ti = carry
            base = j * _LANES
            v = rowbuf[pl.ds(shift + base, _LANES)]
            ci = base + lane
            v = jnp.where(ci < _NROWS, v, _NEG)
            cs, cis = plsc.sort_key_val(v, ci, descending=False)
            keep = (tv > cs) | ((tv == cs) & (ti < cis))
            nv = jnp.where(keep, tv, cs)
            ni = jnp.where(keep, ti, cis)
            return tuple(plsc.sort_key_val(nv, ni, descending=True))

        tv0 = jnp.full((_LANES,), _NEG, jnp.float32)
        ti0 = jnp.zeros((_LANES,), jnp.int32)
        _, ti = lax.fori_loop(0, _CHUNKS, chunk_body, (tv0, ti0))
        stage[...] = ti
        pltpu.sync_copy(stage, out_hbm.at[pl.ds(row * _LANES, _LANES)])
        return 0

    lax.fori_loop(0, nrows, row_body, 0)


def _sc_topk(conv_flat):
    mesh = plsc.VectorSubcoreMesh(
        core_axis_name="c", subcore_axis_name="s",
        num_cores=_NCORES, num_subcores=_NSUB)
    return pl.kernel(
        _topk_body,
        out_type=jax.ShapeDtypeStruct((_NROWS * _LANES,), jnp.int32),
        mesh=mesh,
        scratch_types=[
            pltpu.VMEM((_CBUF,), jnp.float32),
            pltpu.VMEM((_LANES,), jnp.int32),
        ],
        compiler_params=pltpu.CompilerParams(needs_layout_passes=False),
    )(conv_flat)


def kernel(upDown_count_T, W, b):
    x2 = upDown_count_T.reshape(4, _TOT)
    wrow = W.reshape(1, 4).astype(jnp.float32)
    conv = _conv_tc(x2, wrow, b.astype(jnp.float32))
    idx16 = _sc_topk(conv.reshape(_CONVW)).reshape(_NROWS, _LANES)
    row_idx = idx16[:, 1:6].reshape(-1)
    col = jnp.repeat(jnp.arange(_NROWS, dtype=jnp.int32), 5)
    return jnp.stack([row_idx, col], axis=0)


# speedup vs baseline: 1.3561x; 1.3254x over previous
import functools
import jax
import jax.numpy as jnp
from jax import lax
from jax.experimental import pallas as pl
from jax.experimental.pallas import tpu as pltpu
from jax.experimental.pallas import tpu_sc as plsc

_NROWS = 970
_LANES = 16
_NCORES = 2
_NSUB = 16
_NWORK = _NCORES * _NSUB
_CHUNKS = (_NROWS + _LANES - 1) // _LANES
_NPAD = _CHUNKS * _LANES
_OUTW = 1024
_RB = 8
_GRID = _NPAD // _RB
_NEG = float(jnp.finfo(jnp.float32).min)


def _conv_body(b_sm, w_ref, x_ref, o_ref):
    xm = x_ref[...].reshape(4 * _RB, _NROWS)
    res = lax.dot_general(w_ref[...], xm, (((1,), (0,)), ((), ())),
                          preferred_element_type=jnp.float32)
    o_ref[:, pl.ds(0, _NROWS)] = res + b_sm[0]


def _conv_tc(x, w32, b):
    return pl.pallas_call(
        _conv_body,
        grid=(_GRID,),
        in_specs=[
            pl.BlockSpec(memory_space=pltpu.SMEM),
            pl.BlockSpec((_RB, 4 * _RB), lambda i: (0, 0)),
            pl.BlockSpec((4, _RB, _NROWS), lambda i: (0, i, 0)),
        ],
        out_specs=pl.BlockSpec((_RB, _OUTW), lambda i: (i, 0)),
        out_shape=jax.ShapeDtypeStruct((_NPAD, _OUTW), jnp.float32),
    )(b, w32, x)


def _topk_body(conv_hbm, out_hbm, rowbuf, stage):
    cid = lax.axis_index("c")
    sid = lax.axis_index("s")
    wid = sid * _NCORES + cid
    lane = lax.iota(jnp.int32, _LANES)

    nrows = jnp.where(wid < _NROWS - 30 * _NWORK, 31, 30)

    def row_body(t, _):
        row = wid + _NWORK * t
        pltpu.sync_copy(conv_hbm.at[pl.ds(row * _OUTW, _NPAD)], rowbuf)

        def chunk_body(j, carry):
            tv, ti = carry
            base = j * _LANES
            v = rowbuf[pl.ds(base, _LANES)]
            ci = base + lane
            v = jnp.where(ci < _NROWS, v, _NEG)
            cs, cis = plsc.sort_key_val(v, ci, descending=False)
            keep = (tv > cs) | ((tv == cs) & (ti < cis))
            nv = jnp.where(keep, tv, cs)
            ni = jnp.where(keep, ti, cis)
            return tuple(plsc.sort_key_val(nv, ni, descending=True))

        tv0 = jnp.full((_LANES,), _NEG, jnp.float32)
        ti0 = jnp.zeros((_LANES,), jnp.int32)
        _, ti = lax.fori_loop(0, _CHUNKS, chunk_body, (tv0, ti0))
        stage[...] = ti
        pltpu.sync_copy(stage, out_hbm.at[pl.ds(row * _LANES, _LANES)])
        return 0

    lax.fori_loop(0, nrows, row_body, 0)


def _sc_topk(conv2):
    mesh = plsc.VectorSubcoreMesh(
        core_axis_name="c", subcore_axis_name="s",
        num_cores=_NCORES, num_subcores=_NSUB)
    return pl.kernel(
        _topk_body,
        out_type=jax.ShapeDtypeStruct((_NROWS * _LANES,), jnp.int32),
        mesh=mesh,
        scratch_types=[
            pltpu.VMEM((_NPAD,), jnp.float32),
            pltpu.VMEM((_LANES,), jnp.int32),
        ],
        compiler_params=pltpu.CompilerParams(needs_layout_passes=False),
    )(conv2)


def kernel(upDown_count_T, W, b):
    w4 = W.reshape(4).astype(jnp.float32)
    w32 = (jnp.eye(_RB, dtype=jnp.float32)[:, None, :]
           * w4[None, :, None]).reshape(_RB, 4 * _RB)
    conv2 = _conv_tc(upDown_count_T, w32, b.astype(jnp.float32))
    idx16 = _sc_topk(conv2.reshape(_NPAD * _OUTW)).reshape(_NROWS, _LANES)
    row_idx = idx16[:, 1:6].reshape(-1)
    col = jnp.repeat(jnp.arange(_NROWS, dtype=jnp.int32), 5)
    return jnp.stack([row_idx, col], axis=0)


# speedup vs baseline: 2.1486x; 1.5844x over previous
import functools
import jax
import jax.numpy as jnp
from jax import lax
from jax.experimental import pallas as pl
from jax.experimental.pallas import tpu as pltpu
from jax.experimental.pallas import tpu_sc as plsc

_NROWS = 970
_LANES = 16
_NCORES = 2
_NSUB = 16
_NWORK = _NCORES * _NSUB
_CHUNKS = (_NROWS + _LANES - 1) // _LANES
_NPAD = _CHUNKS * _LANES
_OUTW = 1024
_RB = 64
_GRID = -(-_NPAD // _RB)
_NEG = float(jnp.finfo(jnp.float32).min)


def _conv_body(b_sm, w_ref, x_ref, o_ref):
    xm = x_ref[...].reshape(4 * _RB, _NROWS)
    res = lax.dot_general(w_ref[...], xm, (((1,), (0,)), ((), ())),
                          preferred_element_type=jnp.float32)
    o_ref[:, pl.ds(0, _NROWS)] = res + b_sm[0]


def _conv_tc(x, w32, b):
    return pl.pallas_call(
        _conv_body,
        grid=(_GRID,),
        in_specs=[
            pl.BlockSpec(memory_space=pltpu.SMEM),
            pl.BlockSpec((_RB, 4 * _RB), lambda i: (0, 0)),
            pl.BlockSpec((4, _RB, _NROWS), lambda i: (0, i, 0)),
        ],
        out_specs=pl.BlockSpec((_RB, _OUTW), lambda i: (i, 0)),
        out_shape=jax.ShapeDtypeStruct((_NPAD, _OUTW), jnp.float32),
    )(b, w32, x)


def _topk_body(conv_hbm, out_hbm, rowbuf, stage):
    cid = lax.axis_index("c")
    sid = lax.axis_index("s")
    wid = sid * _NCORES + cid
    lane = lax.iota(jnp.int32, _LANES)

    nrows = jnp.where(wid < _NROWS - 30 * _NWORK, 31, 30)

    def row_body(t, _):
        row = wid + _NWORK * t
        pltpu.sync_copy(conv_hbm.at[pl.ds(row * _OUTW, _NPAD)], rowbuf)

        def chunk_body(j, carry):
            tv, ti = carry
            base = j * _LANES
            v = rowbuf[pl.ds(base, _LANES)]
            ci = base + lane
            v = jnp.where(ci < _NROWS, v, _NEG)
            cs, cis = plsc.sort_key_val(v, ci, descending=False)
            keep = (tv > cs) | ((tv == cs) & (ti < cis))
            nv = jnp.where(keep, tv, cs)
            ni = jnp.where(keep, ti, cis)
            return tuple(plsc.sort_key_val(nv, ni, descending=True))

        tv0 = jnp.full((_LANES,), _NEG, jnp.float32)
        ti0 = jnp.zeros((_LANES,), jnp.int32)
        _, ti = lax.fori_loop(0, _CHUNKS, chunk_body, (tv0, ti0))
        stage[...] = ti
        pltpu.sync_copy(stage, out_hbm.at[pl.ds(row * _LANES, _LANES)])
        return 0

    lax.fori_loop(0, nrows, row_body, 0)


def _sc_topk(conv2):
    mesh = plsc.VectorSubcoreMesh(
        core_axis_name="c", subcore_axis_name="s",
        num_cores=_NCORES, num_subcores=_NSUB)
    return pl.kernel(
        _topk_body,
        out_type=jax.ShapeDtypeStruct((_NROWS * _LANES,), jnp.int32),
        mesh=mesh,
        scratch_types=[
            pltpu.VMEM((_NPAD,), jnp.float32),
            pltpu.VMEM((_LANES,), jnp.int32),
        ],
        compiler_params=pltpu.CompilerParams(needs_layout_passes=False),
    )(conv2)


def kernel(upDown_count_T, W, b):
    w4 = W.reshape(4).astype(jnp.float32)
    w32 = (jnp.eye(_RB, dtype=jnp.float32)[:, None, :]
           * w4[None, :, None]).reshape(_RB, 4 * _RB)
    conv2 = _conv_tc(upDown_count_T, w32, b.astype(jnp.float32))
    idx16 = _sc_topk(conv2.reshape(_NPAD * _OUTW)).reshape(_NROWS, _LANES)
    row_idx = idx16[:, 1:6].reshape(-1)
    col = jnp.repeat(jnp.arange(_NROWS, dtype=jnp.int32), 5)
    return jnp.stack([row_idx, col], axis=0)
